# baseline (speedup 1.0000x reference)
import functools

import jax
import jax.numpy as jnp
from jax import lax
from jax.experimental import pallas as pl
from jax.experimental.pallas import tpu as pltpu
from jax.experimental.pallas import tpu_sc as plsc

N = 10000
D = 256
E = 160000

NC, NS = 2, 16
NPAD = 10240
ROWS_PT = NPAD // NS
EPT = E // NS
KC = 80
RC = EPT // KC
NQ = 4
QW = D // NQ
HALF = D // NC
XCH = 80
NCH = ROWS_PT // XCH

_mesh = plsc.VectorSubcoreMesh(core_axis_name="c", subcore_axis_name="s")


def _rsqrt16(v):
    i = plsc.bitcast(v, jnp.int32)
    i = jnp.int32(0x5F3759DF) - lax.shift_right_logical(i, 1)
    y = plsc.bitcast(i, jnp.float32)
    for _ in range(3):
        y = y * (1.5 - 0.5 * v * y * y)
    return y


@functools.partial(
    pl.kernel,
    out_type=(
        jax.ShapeDtypeStruct((NQ, NPAD, QW), jnp.float32),
        jax.ShapeDtypeStruct((NQ, NPAD, QW), jnp.float32),
    ),
    mesh=_mesh,
    compiler_params=pltpu.CompilerParams(use_tc_tiling_on_sc=False,
                                         needs_layout_passes=False),
    scratch_types=[
        pltpu.VMEM((RC, KC), jnp.int32),
        pltpu.VMEM((RC, KC), jnp.int32),
        pltpu.VMEM((KC, 16), jnp.float32),
        pltpu.VMEM((2, KC, QW), jnp.float32),
        pltpu.VMEM((XCH, HALF), jnp.float32),
        pltpu.VMEM((XCH, QW), jnp.float32),
        pltpu.VMEM((XCH, QW), jnp.float32),
        pltpu.VMEM((ROWS_PT, 16), jnp.float32),
        pltpu.VMEM_SHARED((NPAD, 16), jnp.float32),
        pltpu.VMEM_SHARED((NPAD, QW), jnp.float32),
        pltpu.SemaphoreType.DMA((2,)),
        pltpu.SemaphoreType.DMA((2,)),
    ],
)
def _fused_kernel(x, e3, ones16, zeros16, zerosq, agg_out, h_out,
                  sidx, didx, ones_v, rows, xbuf, hbuf0, hbuf1, degbuf,
                  sdeg, sacc, gsem, ssem):
    c = lax.axis_index("c")
    s = lax.axis_index("s")
    sl = pl.ds(s * ROWS_PT, ROWS_PT)

    pltpu.sync_copy(ones16, ones_v)
    pltpu.sync_copy(e3.at[0, s], sidx)
    pltpu.sync_copy(e3.at[1, s], didx)
    pltpu.sync_copy(zeros16.at[sl], sdeg.at[sl])
    pltpu.sync_copy(zerosq.at[sl], sacc.at[sl])
    plsc.subcore_barrier()

    def dstep_src(j, carry):
        pltpu.sync_copy(ones_v, sdeg.at[sidx.at[j]], add=True)
        return carry

    lax.fori_loop(0, RC, dstep_src, 0)
    plsc.subcore_barrier()

    pltpu.sync_copy(sdeg.at[sl], degbuf)

    for ch in range(NCH):
        row0 = s * ROWS_PT + ch * XCH

        @pl.when(row0 < N)
        def _chunk():
            pltpu.sync_copy(
                x.at[pl.ds(row0, XCH), pl.ds(c * HALF, HALF)], xbuf)

            def scale_row(gi, carry):
                dv = degbuf[ch * XCH + gi, :]
                nrm = _rsqrt16(jnp.maximum(dv, 1.0))
                for k in range(8):
                    hk = xbuf[gi, pl.ds(k * 16, 16)] * nrm
                    if k < 4:
                        hbuf0[gi, pl.ds(k * 16, 16)] = hk
                    else:
                        hbuf1[gi, pl.ds((k - 4) * 16, 16)] = hk
                return carry

            lax.fori_loop(0, XCH, scale_row, 0)
            pltpu.sync_copy(hbuf0, h_out.at[c * 2, pl.ds(row0, XCH)])
            pltpu.sync_copy(hbuf1, h_out.at[c * 2 + 1, pl.ds(row0, XCH)])

    plsc.subcore_barrier()

    pltpu.sync_copy(zeros16.at[sl], sdeg.at[sl])
    plsc.subcore_barrier()

    def dstep_dst(j, carry):
        pltpu.sync_copy(ones_v, sdeg.at[didx.at[j]], add=True)
        return carry

    lax.fori_loop(0, RC, dstep_dst, 0)
    plsc.subcore_barrier()
    pltpu.sync_copy(sdeg.at[sl], degbuf)

    for p in range(2):
        q = c * 2 + p
        table = h_out.at[q]
        plsc.subcore_barrier()
        pltpu.async_copy(table.at[sidx.at[0]], rows.at[0], gsem.at[0])

        def step(j, carry):
            slot = lax.rem(j, 2)
            nxt = lax.rem(j + 1, 2)

            @pl.when(j + 1 < RC)
            def _prefetch():
                @pl.when(j >= 1)
                def _wait_prev_scatter():
                    pltpu.make_async_copy(rows.at[nxt],
                                          sacc.at[didx.at[j - 1]],
                                          ssem.at[nxt]).wait()

                pltpu.async_copy(table.at[sidx.at[j + 1]], rows.at[nxt],
                                 gsem.at[nxt])

            pltpu.make_async_copy(table.at[sidx.at[j]], rows.at[slot],
                                  gsem.at[slot]).wait()
            pltpu.async_copy(rows.at[slot], sacc.at[didx.at[j]],
                             ssem.at[slot], add=True)
            return carry

        lax.fori_loop(0, RC, step, 0)
        pltpu.make_async_copy(rows.at[RC % 2], sacc.at[didx.at[RC - 2]],
                              ssem.at[RC % 2]).wait()
        pltpu.make_async_copy(rows.at[(RC - 1) % 2], sacc.at[didx.at[RC - 1]],
                              ssem.at[(RC - 1) % 2]).wait()
        plsc.subcore_barrier()

        for ch in range(NCH):
            r0 = ch * XCH
            pltpu.sync_copy(sacc.at[pl.ds(s * ROWS_PT + r0, XCH)], hbuf0)

            def norm_row(gi, carry):
                dv = degbuf[r0 + gi, :]
                nrm = _rsqrt16(jnp.maximum(dv, 1.0))
                for k in range(4):
                    hbuf1[gi, pl.ds(k * 16, 16)] = (
                        hbuf0[gi, pl.ds(k * 16, 16)] * nrm)
                return carry

            lax.fori_loop(0, XCH, norm_row, 0)
            pltpu.sync_copy(hbuf1,
                            agg_out.at[q, pl.ds(s * ROWS_PT + r0, XCH)])

        @pl.when(p == 0)
        def _rezero():
            pltpu.sync_copy(zerosq.at[sl], sacc.at[sl])


BN = 400


def _gru_body(agg_ref, wgc_ref, bgc_ref, wih_ref, bih_ref, bhh_ref, o_ref):
    gc = bgc_ref[...]
    for q in range(NQ):
        gc = gc + jnp.dot(agg_ref[q], wgc_ref[q * QW:(q + 1) * QW, :],
                          preferred_element_type=jnp.float32)
    gc = jnp.maximum(gc, 0.0)
    gi = lax.dot_general(gc, wih_ref[...], (((1,), (1,)), ((), ())),
                         preferred_element_type=jnp.float32) + bih_ref[...]
    bhh = bhh_ref[...]
    r = jax.nn.sigmoid(gi[:, :D] + bhh[:, :D])
    z = jax.nn.sigmoid(gi[:, D:2 * D] + bhh[:, D:2 * D])
    n = jnp.tanh(gi[:, 2 * D:] + r * bhh[:, 2 * D:])
    o_ref[...] = (1.0 - z) * n


def _gru(agg4, W_gc, b_gc, W_ih, b_ih, b_hh):
    return pl.pallas_call(
        _gru_body,
        grid=(N // BN,),
        in_specs=[
            pl.BlockSpec((NQ, BN, QW), lambda i: (0, i, 0)),
            pl.BlockSpec((D, D), lambda i: (0, 0)),
            pl.BlockSpec((1, D), lambda i: (0, 0)),
            pl.BlockSpec((3 * D, D), lambda i: (0, 0)),
            pl.BlockSpec((1, 3 * D), lambda i: (0, 0)),
            pl.BlockSpec((1, 3 * D), lambda i: (0, 0)),
        ],
        out_specs=pl.BlockSpec((BN, D), lambda i: (i, 0)),
        out_shape=jax.ShapeDtypeStruct((N, D), jnp.float32),
    )(agg4, W_gc, b_gc, W_ih, b_ih, b_hh)


def kernel(edge_index, node_embeddings, W_gc, b_gc, W_ih, b_ih, W_hh, b_hh):
    del W_hh
    e3 = edge_index.reshape(2, NS, RC, KC)
    ones16 = jnp.ones((KC, 16), jnp.float32)
    zeros16 = jnp.zeros((NPAD, 16), jnp.float32)
    zerosq = jnp.zeros((NPAD, QW), jnp.float32)
    agg4, _h4 = _fused_kernel(node_embeddings, e3, ones16, zeros16, zerosq)
    return _gru(agg4, W_gc, b_gc.reshape(1, D), W_ih,
                b_ih.reshape(1, 3 * D), b_hh.reshape(1, 3 * D))


# speedup vs baseline: 1.2338x; 1.2338x over previous
import functools

import jax
import jax.numpy as jnp
from jax import lax
from jax.experimental import pallas as pl
from jax.experimental.pallas import tpu as pltpu
from jax.experimental.pallas import tpu_sc as plsc

N = 10000
D = 256
E = 160000

NC, NS = 2, 16
NPAD = 10240
ROWS_PT = NPAD // NS
EPT = E // NS
KC = 80
RC = EPT // KC
NQ = 4
QW = D // NQ

_mesh = plsc.VectorSubcoreMesh(core_axis_name="c", subcore_axis_name="s")


@functools.partial(
    pl.kernel,
    out_type=jax.ShapeDtypeStruct((NC, NPAD, 8), jnp.float32),
    mesh=_mesh,
    compiler_params=pltpu.CompilerParams(use_tc_tiling_on_sc=False),
    scratch_types=[
        pltpu.VMEM((RC, KC), jnp.int32),
        pltpu.VMEM((KC, 8), jnp.float32),
        pltpu.VMEM_SHARED((NPAD, 8), jnp.float32),
    ],
)
def _deg_kernel(e3, ones8, zeros8, out, idx_v, ones_v, sdeg):
    c = lax.axis_index("c")
    s = lax.axis_index("s")
    sl = pl.ds(s * ROWS_PT, ROWS_PT)
    pltpu.sync_copy(zeros8.at[sl], sdeg.at[sl])
    pltpu.sync_copy(ones8, ones_v)
    pltpu.sync_copy(e3.at[c, s], idx_v)
    plsc.subcore_barrier()

    def step(j, carry):
        pltpu.sync_copy(ones_v, sdeg.at[idx_v.at[j]], add=True)
        return carry

    lax.fori_loop(0, RC, step, 0)
    plsc.subcore_barrier()
    pltpu.sync_copy(sdeg.at[sl], out.at[c, sl])


@functools.partial(
    pl.kernel,
    out_type=jax.ShapeDtypeStruct((NQ, NPAD, QW), jnp.float32),
    mesh=_mesh,
    compiler_params=pltpu.CompilerParams(use_tc_tiling_on_sc=False),
    scratch_types=[
        pltpu.VMEM((RC, KC), jnp.int32),
        pltpu.VMEM((RC, KC), jnp.int32),
        pltpu.VMEM((2, KC, QW), jnp.float32),
        pltpu.VMEM_SHARED((NPAD, QW), jnp.float32),
        pltpu.SemaphoreType.DMA((2,)),
        pltpu.SemaphoreType.DMA((2,)),
    ],
)
def _agg_kernel(h4, e3, zerosq, out, sidx, didx, rows, sacc, gsem, ssem):
    c = lax.axis_index("c")
    s = lax.axis_index("s")
    sl = pl.ds(s * ROWS_PT, ROWS_PT)
    pltpu.sync_copy(e3.at[0, s], sidx)
    pltpu.sync_copy(e3.at[1, s], didx)
    pltpu.sync_copy(zerosq.at[sl], sacc.at[sl])

    for p in range(2):
        q = c * 2 + p
        table = h4.at[q]
        plsc.subcore_barrier()
        pltpu.async_copy(table.at[sidx.at[0]], rows.at[0], gsem.at[0])

        def step(j, carry):
            slot = lax.rem(j, 2)
            nxt = lax.rem(j + 1, 2)

            @pl.when(j + 1 < RC)
            def _prefetch():
                @pl.when(j >= 1)
                def _wait_prev_scatter():
                    pltpu.make_async_copy(rows.at[nxt],
                                          sacc.at[didx.at[j - 1]],
                                          ssem.at[nxt]).wait()

                pltpu.async_copy(table.at[sidx.at[j + 1]], rows.at[nxt],
                                 gsem.at[nxt])

            pltpu.make_async_copy(table.at[sidx.at[j]], rows.at[slot],
                                  gsem.at[slot]).wait()
            pltpu.async_copy(rows.at[slot], sacc.at[didx.at[j]],
                             ssem.at[slot], add=True)
            return carry

        lax.fori_loop(0, RC, step, 0)
        pltpu.make_async_copy(rows.at[RC % 2], sacc.at[didx.at[RC - 2]],
                              ssem.at[RC % 2]).wait()
        pltpu.make_async_copy(rows.at[(RC - 1) % 2], sacc.at[didx.at[RC - 1]],
                              ssem.at[(RC - 1) % 2]).wait()
        plsc.subcore_barrier()
        pltpu.sync_copy(sacc.at[sl], out.at[q, sl])
        if p == 0:
            pltpu.sync_copy(zerosq.at[sl], sacc.at[sl])


BN = 2000


def _scale_body(x_ref, deg_ref, o_ref):
    nrm = lax.rsqrt(jnp.maximum(deg_ref[0, :, 0:1], 1.0))
    h = x_ref[...] * nrm
    for q in range(NQ):
        o_ref[q] = h[:, q * QW:(q + 1) * QW]


def _scale(x, degs8):
    return pl.pallas_call(
        _scale_body,
        grid=(N // BN,),
        in_specs=[
            pl.BlockSpec((BN, D), lambda i: (i, 0)),
            pl.BlockSpec((1, BN, 8), lambda i: (0, i, 0)),
        ],
        out_specs=pl.BlockSpec((NQ, BN, QW), lambda i: (0, i, 0)),
        out_shape=jax.ShapeDtypeStruct((NQ, N, QW), jnp.float32),
    )(x, degs8)


def _gru_body(agg_ref, deg_ref, wgc_ref, bgc_ref, wih_ref, bih_ref, bhh_ref,
              o_ref):
    nd = lax.rsqrt(jnp.maximum(deg_ref[0, :, 0:1], 1.0))
    gc = bgc_ref[...]
    for q in range(NQ):
        gc = gc + jnp.dot(agg_ref[q] * nd, wgc_ref[q * QW:(q + 1) * QW, :],
                          preferred_element_type=jnp.float32)
    gc = jnp.maximum(gc, 0.0)
    gi = lax.dot_general(gc, wih_ref[...], (((1,), (1,)), ((), ())),
                         preferred_element_type=jnp.float32) + bih_ref[...]
    bhh = bhh_ref[...]
    r = jax.nn.sigmoid(gi[:, :D] + bhh[:, :D])
    z = jax.nn.sigmoid(gi[:, D:2 * D] + bhh[:, D:2 * D])
    n = jnp.tanh(gi[:, 2 * D:] + r * bhh[:, 2 * D:])
    o_ref[...] = (1.0 - z) * n


def _gru(agg4, degs8, W_gc, b_gc, W_ih, b_ih, b_hh):
    return pl.pallas_call(
        _gru_body,
        grid=(N // BN,),
        in_specs=[
            pl.BlockSpec((NQ, BN, QW), lambda i: (0, i, 0)),
            pl.BlockSpec((1, BN, 8), lambda i: (1, i, 0)),
            pl.BlockSpec((D, D), lambda i: (0, 0)),
            pl.BlockSpec((1, D), lambda i: (0, 0)),
            pl.BlockSpec((3 * D, D), lambda i: (0, 0)),
            pl.BlockSpec((1, 3 * D), lambda i: (0, 0)),
            pl.BlockSpec((1, 3 * D), lambda i: (0, 0)),
        ],
        out_specs=pl.BlockSpec((BN, D), lambda i: (i, 0)),
        out_shape=jax.ShapeDtypeStruct((N, D), jnp.float32),
    )(agg4, degs8, W_gc, b_gc, W_ih, b_ih, b_hh)


def kernel(edge_index, node_embeddings, W_gc, b_gc, W_ih, b_ih, W_hh, b_hh):
    del W_hh
    e3 = edge_index.reshape(2, NS, RC, KC)
    ones8 = jnp.ones((KC, 8), jnp.float32)
    zeros8 = jnp.zeros((NPAD, 8), jnp.float32)
    zerosq = jnp.zeros((NPAD, QW), jnp.float32)
    degs8 = _deg_kernel(e3, ones8, zeros8)
    h4 = _scale(node_embeddings, degs8)
    agg4 = _agg_kernel(h4, e3, zerosq)
    return _gru(agg4, degs8, W_gc, b_gc.reshape(1, D), W_ih,
                b_ih.reshape(1, 3 * D), b_hh.reshape(1, 3 * D))
